# Initial kernel scaffold; baseline (speedup 1.0000x reference)
import jax
import jax.numpy as jnp
from jax.experimental import pallas as pl


def kernel(x, W1, b1, W2, b2):
    raise NotImplementedError("write your pallas kernel here")


# speedup vs baseline: 1.3060x; 1.3060x over previous
import jax
import jax.numpy as jnp
from jax import lax
from jax.experimental import pallas as pl
from jax.experimental.pallas import tpu as pltpu


def _body(x_ref, w1_ref, b1_ref, w2_ref, b2_ref, out_ref, corr_ref):
    bi = pl.program_id(1)
    nb = pl.num_programs(1)

    xb = x_ref[0]
    t = xb.shape[0]
    d = xb.shape[1]
    inv = 1.0 / (t - 1)

    gram = lax.dot_general(
        xb, xb, (((0,), (0,)), ((), ())),
        preferred_element_type=jnp.float32)
    colsum = jnp.sum(xb, axis=0)
    mean = colsum * (1.0 / t)
    cov = (gram - colsum[:, None] * mean[None, :]) * inv
    rows = lax.broadcasted_iota(jnp.int32, (d, d), 0)
    cols = lax.broadcasted_iota(jnp.int32, (d, d), 1)
    var = jnp.sum(jnp.where(rows == cols, cov, 0.0), axis=0)
    std = jnp.sqrt(var)
    denom = std[:, None] * std[None, :] + 1e-8
    corr = jnp.clip(cov / denom, -1.0, 1.0)
    s = jnp.mean(jnp.abs(corr)) / nb

    @pl.when(bi == 0)
    def _init():
        corr_ref[...] = jnp.zeros_like(corr_ref)

    corr_ref[...] += jnp.broadcast_to(s, (1, 1, 1))

    w1 = w1_ref[0]
    b1 = b1_ref[0]
    w2 = w2_ref[0]
    b2 = b2_ref[0]
    h = jnp.maximum(
        jnp.dot(xb, w1, preferred_element_type=jnp.float32) + b1, 0.0)
    o = jnp.dot(h, w2, preferred_element_type=jnp.float32) + b2
    out_ref[...] = o[None]


def kernel(x, W1, b1, W2, b2):
    B, N, d = x.shape
    G = W1.shape[0]
    n = N // G
    H = W1.shape[2]

    b1r = b1.reshape(G, 1, H)
    b2r = b2.reshape(G, 1, 1)

    out, corr = pl.pallas_call(
        _body,
        grid=(G, B),
        in_specs=[
            pl.BlockSpec((1, n, d), lambda g, b: (b, g, 0)),
            pl.BlockSpec((1, d, H), lambda g, b: (g, 0, 0)),
            pl.BlockSpec((1, 1, H), lambda g, b: (g, 0, 0)),
            pl.BlockSpec((1, H, 1), lambda g, b: (g, 0, 0)),
            pl.BlockSpec((1, 1, 1), lambda g, b: (g, 0, 0)),
        ],
        out_specs=[
            pl.BlockSpec((1, n, 1), lambda g, b: (b, g, 0)),
            pl.BlockSpec((1, 1, 1), lambda g, b: (g, 0, 0)),
        ],
        out_shape=[
            jax.ShapeDtypeStruct((B, N, 1), jnp.float32),
            jax.ShapeDtypeStruct((G, 1, 1), jnp.float32),
        ],
        compiler_params=pltpu.CompilerParams(
            dimension_semantics=("parallel", "arbitrary")),
    )(x, W1, b1r, W2, b2r)
    return (out, corr.reshape(G))


# speedup vs baseline: 2.0766x; 1.5901x over previous
import jax
import jax.numpy as jnp
from jax import lax
from jax.experimental import pallas as pl
from jax.experimental.pallas import tpu as pltpu


def _body(x_ref, w1_ref, b1_ref, w2_ref, b2_ref, out_ref, corr_ref):
    bi = pl.program_id(1)
    nb = pl.num_programs(1)

    xb = x_ref[0]
    t = xb.shape[0]
    d = xb.shape[1]
    inv = 1.0 / (t - 1)

    w1 = w1_ref[0]
    b1 = b1_ref[0]
    w2 = w2_ref[0]
    b2 = b2_ref[0]
    ht = jnp.maximum(
        lax.dot_general(w1, xb, (((0,), (1,)), ((), ())),
                        preferred_element_type=jnp.float32) + b1, 0.0)
    ot = jnp.sum(ht * w2, axis=0, keepdims=True) + b2
    out_ref[...] = ot[None]

    gram = lax.dot_general(
        xb, xb, (((0,), (0,)), ((), ())),
        preferred_element_type=jnp.float32)
    colsum = jnp.sum(xb, axis=0)
    mean = colsum * (1.0 / t)
    cov = (gram - colsum[:, None] * mean[None, :]) * inv
    rows = lax.broadcasted_iota(jnp.int32, (d, d), 0)
    cols = lax.broadcasted_iota(jnp.int32, (d, d), 1)
    var = jnp.sum(jnp.where(rows == cols, cov, 0.0), axis=0)
    std = jnp.sqrt(var)
    denom = std[:, None] * std[None, :] + 1e-8
    corr = jnp.clip(cov / denom, -1.0, 1.0)
    s = jnp.mean(jnp.abs(corr)) / nb

    @pl.when(bi == 0)
    def _init():
        corr_ref[...] = jnp.zeros_like(corr_ref)

    corr_ref[...] += jnp.broadcast_to(s, (1, 1, 1))


def kernel(x, W1, b1, W2, b2):
    B, N, d = x.shape
    G = W1.shape[0]
    n = N // G
    H = W1.shape[2]

    b1r = b1.reshape(G, H, 1)
    b2r = b2.reshape(G, 1, 1)

    out, corr = pl.pallas_call(
        _body,
        grid=(G, B),
        in_specs=[
            pl.BlockSpec((1, n, d), lambda g, b: (b, g, 0)),
            pl.BlockSpec((1, d, H), lambda g, b: (g, 0, 0)),
            pl.BlockSpec((1, H, 1), lambda g, b: (g, 0, 0)),
            pl.BlockSpec((1, H, 1), lambda g, b: (g, 0, 0)),
            pl.BlockSpec((1, 1, 1), lambda g, b: (g, 0, 0)),
        ],
        out_specs=[
            pl.BlockSpec((1, 1, n), lambda g, b: (b * G + g, 0, 0)),
            pl.BlockSpec((1, 1, 1), lambda g, b: (g, 0, 0)),
        ],
        out_shape=[
            jax.ShapeDtypeStruct((B * G, 1, n), jnp.float32),
            jax.ShapeDtypeStruct((G, 1, 1), jnp.float32),
        ],
        compiler_params=pltpu.CompilerParams(
            dimension_semantics=("parallel", "arbitrary")),
    )(x, W1, b1r, W2, b2r)
    return (out.reshape(B, N, 1), corr.reshape(G))
